# Initial kernel scaffold; baseline (speedup 1.0000x reference)
import jax
import jax.numpy as jnp
from jax.experimental import pallas as pl


def kernel(word, remainder, word_table, rem_table, W, b):
    raise NotImplementedError("write your pallas kernel here")


# speedup vs baseline: 6.1983x; 6.1983x over previous
import functools

import jax
import jax.numpy as jnp
from jax import lax
from jax.experimental import pallas as pl
from jax.experimental.pallas import tpu as pltpu
from jax.experimental.pallas import tpu_sc as plsc

D = 128
_NC, _NS = 2, 16
_NW = _NC * _NS
_CHUNK = 128


def _proj_body(x_ref, wt_ref, b_ref, o_ref):
    o_ref[...] = lax.dot_general(
        x_ref[...], wt_ref[...], (((1,), (1,)), ((), ())),
        preferred_element_type=jnp.float32,
    ) + b_ref[...]


def _project(table, wt, bias, row_block):
    rows = table.shape[0]
    grid = rows // row_block
    return pl.pallas_call(
        _proj_body,
        grid=(grid,),
        in_specs=[
            pl.BlockSpec((row_block, D), lambda i: (i, 0)),
            pl.BlockSpec((D, D), lambda i: (0, 0)),
            pl.BlockSpec((1, D), lambda i: (0, 0)),
        ],
        out_specs=pl.BlockSpec((row_block, D), lambda i: (i, 0)),
        out_shape=jax.ShapeDtypeStruct((rows, D), jnp.float32),
    )(table, wt, bias)


def _gather_add_body(widx_hbm, ridx_hbm, wtab_hbm, rtab_hbm, out_hbm,
                     widx_v, ridx_v, rows_w, rows_r, sem_w, sem_r):
    n = out_hbm.shape[0]
    rpw = n // _NW
    nchunk = rpw // _CHUNK
    wid = lax.axis_index("s") * _NC + lax.axis_index("c")
    base0 = wid * rpw

    @pl.loop(0, nchunk)
    def _chunk(ci):
        base = base0 + ci * _CHUNK
        pltpu.sync_copy(widx_hbm.at[pl.ds(base, _CHUNK)], widx_v)
        pltpu.sync_copy(ridx_hbm.at[pl.ds(base, _CHUNK)], ridx_v)
        cw = pltpu.async_copy(wtab_hbm.at[widx_v], rows_w, sem_w)
        cr = pltpu.async_copy(rtab_hbm.at[ridx_v], rows_r, sem_r)
        cw.wait()
        cr.wait()

        @pl.loop(0, _CHUNK)
        def _row(r):
            for g in range(D // 16):
                sl = pl.ds(g * 16, 16)
                rows_w[r, sl] = rows_w[r, sl] + rows_r[r, sl]

        pltpu.sync_copy(rows_w, out_hbm.at[pl.ds(base, _CHUNK)])


def _gather_add(widx, ridx, wtab, rtab):
    n = widx.shape[0]
    mesh = plsc.VectorSubcoreMesh(core_axis_name="c", subcore_axis_name="s")
    fn = pl.kernel(
        _gather_add_body,
        out_type=jax.ShapeDtypeStruct((n, D), jnp.float32),
        mesh=mesh,
        scratch_types=[
            pltpu.VMEM((_CHUNK,), jnp.int32),
            pltpu.VMEM((_CHUNK,), jnp.int32),
            pltpu.VMEM((_CHUNK, D), jnp.float32),
            pltpu.VMEM((_CHUNK, D), jnp.float32),
            pltpu.SemaphoreType.DMA,
            pltpu.SemaphoreType.DMA,
        ],
    )
    return fn(widx, ridx, wtab, rtab)


@jax.jit
def kernel(word, remainder, word_table, rem_table, W, b):
    bsz, seq = word.shape
    word_proj = _project(word_table, W[:, :D], jnp.zeros((1, D), jnp.float32),
                         row_block=2000)
    rem_proj = _project(rem_table, W[:, D:], b.reshape(1, D), row_block=512)
    out = _gather_add(word.reshape(-1), remainder.reshape(-1),
                      word_proj, rem_proj)
    return out.reshape(bsz, seq, D)


# speedup vs baseline: 8.6328x; 1.3928x over previous
import functools

import jax
import jax.numpy as jnp
from jax import lax
from jax.experimental import pallas as pl
from jax.experimental.pallas import tpu as pltpu
from jax.experimental.pallas import tpu_sc as plsc

D = 128
_NC, _NS = 2, 16
_NW = _NC * _NS
_CHUNK = 80
_NBUF = 2


def _proj_body(x_ref, wt_ref, b_ref, o_ref):
    o_ref[...] = lax.dot_general(
        x_ref[...], wt_ref[...], (((1,), (1,)), ((), ())),
        preferred_element_type=jnp.float32,
    ) + b_ref[...]


def _project(table, wt, bias, row_block):
    rows = table.shape[0]
    grid = rows // row_block
    return pl.pallas_call(
        _proj_body,
        grid=(grid,),
        in_specs=[
            pl.BlockSpec((row_block, D), lambda i: (i, 0)),
            pl.BlockSpec((D, D), lambda i: (0, 0)),
            pl.BlockSpec((1, D), lambda i: (0, 0)),
        ],
        out_specs=pl.BlockSpec((row_block, D), lambda i: (i, 0)),
        out_shape=jax.ShapeDtypeStruct((rows, D), jnp.float32),
    )(table, wt, bias)


def _gather_add_body(widx_hbm, ridx_hbm, wtab_hbm, rtab_hbm, out_hbm,
                     widx_all, ridx_all, rows_w, rows_r, rows_o,
                     sem_g0, sem_g1, sem_wb0, sem_wb1):
    n = out_hbm.shape[0]
    rpw = n // _NW
    nchunk = rpw // _CHUNK
    wid = lax.axis_index("s") * _NC + lax.axis_index("c")
    base0 = wid * rpw
    sem_g = (sem_g0, sem_g1)
    sem_wb = (sem_wb0, sem_wb1)

    pltpu.sync_copy(widx_hbm.at[pl.ds(base0, rpw)], widx_all)
    pltpu.sync_copy(ridx_hbm.at[pl.ds(base0, rpw)], ridx_all)

    def _issue_gathers(ci, b):
        pltpu.async_copy(wtab_hbm.at[widx_all.at[pl.ds(ci * _CHUNK, _CHUNK)]],
                         rows_w.at[b], sem_g[b])
        pltpu.async_copy(rtab_hbm.at[ridx_all.at[pl.ds(ci * _CHUNK, _CHUNK)]],
                         rows_r.at[b], sem_g[b])

    def _wait_gathers(ci, b):
        pltpu.make_async_copy(
            wtab_hbm.at[widx_all.at[pl.ds(ci * _CHUNK, _CHUNK)]],
            rows_w.at[b], sem_g[b]).wait()
        pltpu.make_async_copy(
            rtab_hbm.at[ridx_all.at[pl.ds(ci * _CHUNK, _CHUNK)]],
            rows_r.at[b], sem_g[b]).wait()

    for b in range(_NBUF):
        _issue_gathers(b, b)

    @pl.loop(0, nchunk, step=_NBUF)
    def _sweep(ci0):
        for b in range(_NBUF):
            ci = ci0 + b
            base = base0 + ci * _CHUNK
            _wait_gathers(ci, b)

            @pl.when(ci >= _NBUF)
            def _():
                pltpu.make_async_copy(
                    rows_o.at[b],
                    out_hbm.at[pl.ds(base - _NBUF * _CHUNK, _CHUNK)],
                    sem_wb[b]).wait()

            @pl.loop(0, _CHUNK)
            def _row(r):
                for g in range(D // 16):
                    sl = pl.ds(g * 16, 16)
                    rows_o[b, r, sl] = rows_w[b, r, sl] + rows_r[b, r, sl]

            pltpu.async_copy(rows_o.at[b], out_hbm.at[pl.ds(base, _CHUNK)],
                             sem_wb[b])

            nxt = ci + _NBUF
            @pl.when(nxt < nchunk)
            def _():
                _issue_gathers(nxt, b)

    for b in range(_NBUF):
        last = nchunk - _NBUF + b
        pltpu.make_async_copy(
            rows_o.at[b], out_hbm.at[pl.ds(base0 + last * _CHUNK, _CHUNK)],
            sem_wb[b]).wait()


def _gather_add(widx, ridx, wtab, rtab):
    n = widx.shape[0]
    rpw = n // _NW
    mesh = plsc.VectorSubcoreMesh(core_axis_name="c", subcore_axis_name="s")
    fn = pl.kernel(
        _gather_add_body,
        out_type=jax.ShapeDtypeStruct((n, D), jnp.float32),
        mesh=mesh,
        scratch_types=[
            pltpu.VMEM((rpw,), jnp.int32),
            pltpu.VMEM((rpw,), jnp.int32),
            pltpu.VMEM((_NBUF, _CHUNK, D), jnp.float32),
            pltpu.VMEM((_NBUF, _CHUNK, D), jnp.float32),
            pltpu.VMEM((_NBUF, _CHUNK, D), jnp.float32),
            pltpu.SemaphoreType.DMA,
            pltpu.SemaphoreType.DMA,
            pltpu.SemaphoreType.DMA,
            pltpu.SemaphoreType.DMA,
        ],
    )
    return fn(widx, ridx, wtab, rtab)


@jax.jit
def kernel(word, remainder, word_table, rem_table, W, b):
    bsz, seq = word.shape
    word_proj = _project(word_table, W[:, :D], jnp.zeros((1, D), jnp.float32),
                         row_block=2000)
    rem_proj = _project(rem_table, W[:, D:], b.reshape(1, D), row_block=512)
    out = _gather_add(word.reshape(-1), remainder.reshape(-1),
                      word_proj, rem_proj)
    return out.reshape(bsz, seq, D)
